# Initial kernel scaffold; baseline (speedup 1.0000x reference)
import jax
import jax.numpy as jnp
from jax.experimental import pallas as pl


def kernel(dataTokens, embed, Wih0f, Whh0f, bih0f, bhh0f, Wih0b, Whh0b, bih0b, bhh0b, Wih1f, Whh1f, bih1f, bhh1f, Wih1b, Whh1b, bih1b, bhh1b, Wih2f, Whh2f, bih2f, bhh2f, Wih2b, Whh2b, bih2b, bhh2b, W1, b1, W11, b11, W2, b2):
    raise NotImplementedError("write your pallas kernel here")


# speedup vs baseline: 6.2311x; 6.2311x over previous
import functools

import jax
import jax.numpy as jnp
from jax import lax
from jax.experimental import pallas as pl
from jax.experimental.pallas import tpu as pltpu
from jax.experimental.pallas import tpu_sc as plsc

B = 64
L = 50
BL = B * L
H = 200
HP = 256
G3 = 3 * HP
E = 100
EP = 128
BL_PAD = 3328


def _sc_gather(table, idx):
    info = plsc.get_sparse_core_info()
    nc, ns = info.num_cores, info.num_subcores
    nw = nc * ns
    b_per_w = BL_PAD // nw

    mesh = plsc.VectorSubcoreMesh(core_axis_name="c", subcore_axis_name="s")

    @functools.partial(
        pl.kernel,
        mesh=mesh,
        out_type=jax.ShapeDtypeStruct((BL_PAD, EP), jnp.float32),
        scratch_types=[
            pltpu.VMEM((b_per_w,), jnp.int32),
            pltpu.VMEM((b_per_w, EP), jnp.float32),
            pltpu.SemaphoreType.DMA,
        ],
    )
    def gather_kernel(table_hbm, idx_hbm, out_hbm, idx_v, rows_v, sem):
        wid = lax.axis_index("s") * nc + lax.axis_index("c")
        base = wid * b_per_w
        pltpu.sync_copy(idx_hbm.at[pl.ds(base, b_per_w)], idx_v)
        pltpu.async_copy(table_hbm.at[idx_v], rows_v, sem).wait()
        pltpu.sync_copy(rows_v, out_hbm.at[pl.ds(base, b_per_w)])

    return gather_kernel(table, idx)


def _prep_ih(W):
    cin = W.shape[1]
    Wg = W.reshape(3, H, cin)
    Wg = jnp.pad(Wg, ((0, 0), (0, HP - H), (0, 0)))
    Wt = Wg.reshape(G3, cin).T
    if cin == E:
        return jnp.pad(Wt, ((0, EP - E), (0, 0)))
    z = jnp.zeros((HP - H, G3), Wt.dtype)
    return jnp.concatenate([Wt[0:H], z, Wt[H:2 * H], z], axis=0)


def _prep_hh(W):
    Wg = W.reshape(3, H, H)
    Wg = jnp.pad(Wg, ((0, 0), (0, HP - H), (0, 0)))
    Wt = Wg.reshape(G3, H).T
    return jnp.pad(Wt, ((0, HP - H), (0, 0)))


def _prep_bias(b):
    return jnp.pad(b.reshape(3, H), ((0, 0), (0, HP - H))).reshape(1, G3)


def _tc_body(x_ref,
             wihf0, wihb0, whhf0, whhb0, bihf0, bihb0, bhhf0, bhhb0,
             wihf1, wihb1, whhf1, whhb1, bihf1, bihb1, bhhf1, bhhb1,
             wihf2, wihb2, whhf2, whhb2, bihf2, bihb2, bhhf2, bhhb2,
             w1t, b1, w11t, b11, w2t, b2,
             out_ref, gf, gb, y1, y2):
    wih = ((wihf0, wihb0), (wihf1, wihb1), (wihf2, wihb2))
    whh = ((whhf0, whhb0), (whhf1, whhb1), (whhf2, whhb2))
    bih = ((bihf0, bihb0), (bihf1, bihb1), (bihf2, bihb2))
    bhh = ((bhhf0, bhhb0), (bhhf1, bhhb1), (bhhf2, bhhb2))
    xin = (x_ref, y1, y2)
    yout = (y1, y2, None)

    finals = []
    for l in range(3):
        wf = wih[l][0][...]
        wb = wih[l][1][...]
        bf = bih[l][0][...]
        bb = bih[l][1][...]
        nch = 8
        rows = BL // nch
        for c in range(nch):
            xs = xin[l][c * rows:(c + 1) * rows, :]
            gf[c * rows:(c + 1) * rows, :] = jnp.dot(xs, wf) + bf
            gb[c * rows:(c + 1) * rows, :] = jnp.dot(xs, wb) + bb

        whf = whh[l][0][...]
        whb = whh[l][1][...]
        bhf = bhh[l][0][...]
        bhb = bhh[l][1][...]
        ydst = yout[l]

        def step(t, h, whf=whf, whb=whb, bhf=bhf, bhb=bhb, ydst=ydst):
            gif = gf[pl.ds(t * B, B), :]
            gib = gb[pl.ds((L - 1) * B - t * B, B), :]
            gi = jnp.concatenate([gif, gib], axis=0)
            ghf = jnp.dot(h[0:B], whf) + bhf
            ghb = jnp.dot(h[B:2 * B], whb) + bhb
            gh = jnp.concatenate([ghf, ghb], axis=0)
            rz = jax.nn.sigmoid(gi[:, 0:2 * HP] + gh[:, 0:2 * HP])
            r = rz[:, 0:HP]
            z = rz[:, HP:2 * HP]
            n = jnp.tanh(gi[:, 2 * HP:G3] + r * gh[:, 2 * HP:G3])
            hn = (1.0 - z) * n + z * h
            if ydst is not None:
                ydst[pl.ds(t * B, B), 0:HP] = hn[0:B]
                ydst[pl.ds((L - 1) * B - t * B, B), HP:2 * HP] = hn[B:2 * B]
            return hn

        h = lax.fori_loop(0, L, step, jnp.zeros((2 * B, HP), jnp.float32))
        finals.append(h[0:B])
        finals.append(h[B:2 * B])

    x1c = jnp.concatenate(finals, axis=1)
    h1 = jnp.maximum(jnp.dot(x1c, w1t[...]) + b1[...], 0.0)
    h2 = jnp.maximum(jnp.dot(h1, w11t[...]) + b11[...], 0.0)
    out_ref[...] = jnp.maximum(jnp.dot(h2, w2t[...]) + b2[...], 0.0)


def _tc_forward(x, args):
    return pl.pallas_call(
        _tc_body,
        out_shape=jax.ShapeDtypeStruct((B, 128), jnp.float32),
        scratch_shapes=[
            pltpu.VMEM((BL, G3), jnp.float32),
            pltpu.VMEM((BL, G3), jnp.float32),
            pltpu.VMEM((BL, 2 * HP), jnp.float32),
            pltpu.VMEM((BL, 2 * HP), jnp.float32),
        ],
    )(x, *args)


def kernel(dataTokens, embed,
           Wih0f, Whh0f, bih0f, bhh0f, Wih0b, Whh0b, bih0b, bhh0b,
           Wih1f, Whh1f, bih1f, bhh1f, Wih1b, Whh1b, bih1b, bhh1b,
           Wih2f, Whh2f, bih2f, bhh2f, Wih2b, Whh2b, bih2b, bhh2b,
           W1, b1, W11, b11, W2, b2):
    idx = dataTokens.T.reshape(-1).astype(jnp.int32)
    idx = jnp.pad(idx, (0, BL_PAD - BL))
    table = jnp.pad(embed, ((0, 0), (0, EP - E)))
    rows = _sc_gather(table, idx)
    x = rows[0:BL]

    prepped = []
    for (Wf, Uf, bf, cf, Wb, Ub, bb, cb) in (
        (Wih0f, Whh0f, bih0f, bhh0f, Wih0b, Whh0b, bih0b, bhh0b),
        (Wih1f, Whh1f, bih1f, bhh1f, Wih1b, Whh1b, bih1b, bhh1b),
        (Wih2f, Whh2f, bih2f, bhh2f, Wih2b, Whh2b, bih2b, bhh2b),
    ):
        prepped += [_prep_ih(Wf), _prep_ih(Wb), _prep_hh(Uf), _prep_hh(Ub),
                    _prep_bias(bf), _prep_bias(bb), _prep_bias(cf), _prep_bias(cb)]

    w1t = W1[:, 2000:].reshape(1000, 6, H)
    w1t = jnp.pad(w1t, ((0, 0), (0, 0), (0, HP - H))).reshape(1000, 6 * HP).T
    w2t = jnp.pad(W2.T, ((0, 0), (0, 128 - 4)))
    b2p = jnp.pad(b2, (0, 128 - 4)).reshape(1, 128)
    prepped += [w1t, b1.reshape(1, 1000), W11.T, b11.reshape(1, 500), w2t, b2p]

    out = _tc_forward(x, prepped)
    return out[:, 0:4]


# speedup vs baseline: 6.5843x; 1.0567x over previous
import functools

import jax
import jax.numpy as jnp
from jax import lax
from jax.experimental import pallas as pl
from jax.experimental.pallas import tpu as pltpu
from jax.experimental.pallas import tpu_sc as plsc

B = 64
L = 50
BL = B * L
H = 200
HP = 256
G3 = 3 * HP
E = 100
EP = 128
BL_PAD = 3328


def _sc_gather(table, idx):
    info = plsc.get_sparse_core_info()
    nc, ns = info.num_cores, info.num_subcores
    nw = nc * ns
    b_per_w = BL_PAD // nw

    mesh = plsc.VectorSubcoreMesh(core_axis_name="c", subcore_axis_name="s")

    @functools.partial(
        pl.kernel,
        mesh=mesh,
        out_type=jax.ShapeDtypeStruct((BL_PAD, EP), jnp.float32),
        scratch_types=[
            pltpu.VMEM((b_per_w,), jnp.int32),
            pltpu.VMEM((b_per_w, EP), jnp.float32),
            pltpu.SemaphoreType.DMA,
        ],
    )
    def gather_kernel(table_hbm, idx_hbm, out_hbm, idx_v, rows_v, sem):
        wid = lax.axis_index("s") * nc + lax.axis_index("c")
        base = wid * b_per_w
        pltpu.sync_copy(idx_hbm.at[pl.ds(base, b_per_w)], idx_v)
        pltpu.async_copy(table_hbm.at[idx_v], rows_v, sem).wait()
        pltpu.sync_copy(rows_v, out_hbm.at[pl.ds(base, b_per_w)])

    return gather_kernel(table, idx)


def _prep_ih(W):
    cin = W.shape[1]
    Wg = W.reshape(3, H, cin)
    Wg = jnp.pad(Wg, ((0, 0), (0, HP - H), (0, 0)))
    Wt = Wg.reshape(G3, cin).T
    if cin == E:
        return jnp.pad(Wt, ((0, EP - E), (0, 0)))
    z = jnp.zeros((HP - H, G3), Wt.dtype)
    return jnp.concatenate([Wt[0:H], z, Wt[H:2 * H], z], axis=0)


def _prep_hh(W):
    Wg = W.reshape(3, H, H)
    Wg = jnp.pad(Wg, ((0, 0), (0, HP - H), (0, 0)))
    Wt = Wg.reshape(G3, H).T
    return jnp.pad(Wt, ((0, HP - H), (0, 0)))


def _prep_bias(b):
    return jnp.pad(b.reshape(3, H), ((0, 0), (0, HP - H))).reshape(1, G3)


def _bdot(a, b):
    return lax.dot(a.astype(jnp.bfloat16), b, preferred_element_type=jnp.float32)


def _tc_body(x_ref,
             wihf0, wihb0, whhf0, whhb0, bihf0, bihb0, bhhf0, bhhb0,
             wihf1, wihb1, whhf1, whhb1, bihf1, bihb1, bhhf1, bhhb1,
             wihf2, wihb2, whhf2, whhb2, bihf2, bihb2, bhhf2, bhhb2,
             w1t, b1, w11t, b11, w2t, b2,
             out_ref, gf, gb, y1, y2):
    wih = ((wihf0, wihb0), (wihf1, wihb1), (wihf2, wihb2))
    whh = ((whhf0, whhb0), (whhf1, whhb1), (whhf2, whhb2))
    bih = ((bihf0, bihb0), (bihf1, bihb1), (bihf2, bihb2))
    bhh = ((bhhf0, bhhb0), (bhhf1, bhhb1), (bhhf2, bhhb2))
    xin = (x_ref, y1, y2)
    yout = (y1, y2, None)

    finals = []
    for l in range(3):
        wf = wih[l][0][...]
        wb = wih[l][1][...]
        bf = bih[l][0][...]
        bb = bih[l][1][...]
        nch = 8
        rows = BL // nch
        for c in range(nch):
            xs = xin[l][c * rows:(c + 1) * rows, :]
            gf[c * rows:(c + 1) * rows, :] = _bdot(xs, wf) + bf
            gb[c * rows:(c + 1) * rows, :] = _bdot(xs, wb) + bb

        whf = whh[l][0][...]
        whb = whh[l][1][...]
        bhf = bhh[l][0][...]
        bhb = bhh[l][1][...]
        ydst = yout[l]

        def step(t, h, whf=whf, whb=whb, bhf=bhf, bhb=bhb, ydst=ydst):
            gif = gf[pl.ds(t * B, B), :]
            gib = gb[pl.ds((L - 1) * B - t * B, B), :]
            gi = jnp.concatenate([gif, gib], axis=0)
            hb16 = h.astype(jnp.bfloat16)
            ghf = _bdot(hb16[0:B], whf) + bhf
            ghb = _bdot(hb16[B:2 * B], whb) + bhb
            gh = jnp.concatenate([ghf, ghb], axis=0)
            rz = jax.nn.sigmoid(gi[:, 0:2 * HP] + gh[:, 0:2 * HP])
            r = rz[:, 0:HP]
            z = rz[:, HP:2 * HP]
            n = jnp.tanh(gi[:, 2 * HP:G3] + r * gh[:, 2 * HP:G3])
            hn = (1.0 - z) * n + z * h
            if ydst is not None:
                hnb = hn.astype(jnp.bfloat16)
                ydst[pl.ds(t * B, B), 0:HP] = hnb[0:B]
                ydst[pl.ds((L - 1) * B - t * B, B), HP:2 * HP] = hnb[B:2 * B]
            return hn

        h = lax.fori_loop(0, L, step, jnp.zeros((2 * B, HP), jnp.float32))
        finals.append(h[0:B])
        finals.append(h[B:2 * B])

    x1c = jnp.concatenate(finals, axis=1)
    h1 = jnp.maximum(jnp.dot(x1c, w1t[...]) + b1[...], 0.0)
    h2 = jnp.maximum(jnp.dot(h1, w11t[...]) + b11[...], 0.0)
    out_ref[...] = jnp.maximum(jnp.dot(h2, w2t[...]) + b2[...], 0.0)


def _tc_forward(x, args):
    return pl.pallas_call(
        _tc_body,
        out_shape=jax.ShapeDtypeStruct((B, 128), jnp.float32),
        scratch_shapes=[
            pltpu.VMEM((BL, G3), jnp.float32),
            pltpu.VMEM((BL, G3), jnp.float32),
            pltpu.VMEM((BL, 2 * HP), jnp.bfloat16),
            pltpu.VMEM((BL, 2 * HP), jnp.bfloat16),
        ],
    )(x, *args)


def kernel(dataTokens, embed,
           Wih0f, Whh0f, bih0f, bhh0f, Wih0b, Whh0b, bih0b, bhh0b,
           Wih1f, Whh1f, bih1f, bhh1f, Wih1b, Whh1b, bih1b, bhh1b,
           Wih2f, Whh2f, bih2f, bhh2f, Wih2b, Whh2b, bih2b, bhh2b,
           W1, b1, W11, b11, W2, b2):
    idx = dataTokens.T.reshape(-1).astype(jnp.int32)
    idx = jnp.pad(idx, (0, BL_PAD - BL))
    table = jnp.pad(embed, ((0, 0), (0, EP - E)))
    rows = _sc_gather(table, idx)
    x = rows[0:BL].astype(jnp.bfloat16)

    prepped = []
    for (Wf, Uf, bf, cf, Wb, Ub, bb, cb) in (
        (Wih0f, Whh0f, bih0f, bhh0f, Wih0b, Whh0b, bih0b, bhh0b),
        (Wih1f, Whh1f, bih1f, bhh1f, Wih1b, Whh1b, bih1b, bhh1b),
        (Wih2f, Whh2f, bih2f, bhh2f, Wih2b, Whh2b, bih2b, bhh2b),
    ):
        bft = jnp.bfloat16
        prepped += [_prep_ih(Wf).astype(bft), _prep_ih(Wb).astype(bft),
                    _prep_hh(Uf).astype(bft), _prep_hh(Ub).astype(bft),
                    _prep_bias(bf), _prep_bias(bb), _prep_bias(cf), _prep_bias(cb)]

    w1t = W1[:, 2000:].reshape(1000, 6, H)
    w1t = jnp.pad(w1t, ((0, 0), (0, 0), (0, HP - H))).reshape(1000, 6 * HP).T
    w2t = jnp.pad(W2.T, ((0, 0), (0, 128 - 4)))
    b2p = jnp.pad(b2, (0, 128 - 4)).reshape(1, 128)
    prepped += [w1t, b1.reshape(1, 1000), W11.T, b11.reshape(1, 500), w2t, b2p]

    out = _tc_forward(x, prepped)
    return out[:, 0:4]
